# Initial kernel scaffold; baseline (speedup 1.0000x reference)
import jax
import jax.numpy as jnp
from jax.experimental import pallas as pl


def kernel(pred0, pred1, pred2, gt_bbox):
    raise NotImplementedError("write your pallas kernel here")


# speedup vs baseline: 3.8341x; 3.8341x over previous
import functools

import jax
import jax.numpy as jnp
from jax.experimental import pallas as pl
from jax.experimental.pallas import tpu as pltpu

_GRIDS = (13, 26, 52)
_A = 3
_NGT = 10
_NC = 80
_THR = 0.5
_WHS = (
    ((3.625, 2.8125), (4.875, 6.1875), (11.65625, 10.1875)),
    ((1.875, 3.8125), (3.875, 2.8125), (3.6875, 7.4375)),
    ((1.25, 1.625), (2.0, 3.75), (4.125, 2.875)),
)


def _scale_terms(p_ref, gts, grid, whs):
    geo = []
    for (x1, y1, x2, y2, cf) in gts:
        gx1, gy1, gx2, gy2 = x1 * grid, y1 * grid, x2 * grid, y2 * grid
        cx = (gx1 + gx2) * 0.5
        cy = (gy1 + gy2) * 0.5
        w = gx2 - gx1
        h = gy2 - gy1
        area = w * h
        geo.append((gx1, gy1, gx2, gy2, cx, cy, w, h, area, cf))

    rows = jax.lax.broadcasted_iota(jnp.int32, (grid, grid), 0).astype(jnp.float32)
    cols = jax.lax.broadcasted_iota(jnp.int32, (grid, grid), 1).astype(jnp.float32)
    lcn = jnp.float32(0.0)
    for a, (wa, ha) in enumerate(whs):
        ax1 = cols + (0.5 - wa * 0.5)
        ax2 = cols + (0.5 + wa * 0.5)
        ay1 = rows + (0.5 - ha * 0.5)
        ay2 = rows + (0.5 + ha * 0.5)
        area_a = wa * ha
        any_ge = None
        for (gx1, gy1, gx2, gy2, cx, cy, w, h, area, cf) in geo:
            ix = jnp.maximum(jnp.minimum(ax2, gx2) - jnp.maximum(ax1, gx1), 0.0)
            iy = jnp.maximum(jnp.minimum(ay2, gy2) - jnp.maximum(ay1, gy1), 0.0)
            inter = ix * iy
            union = jnp.maximum(area_a + area - inter, 1e-16)
            ge = (inter / union) >= _THR
            any_ge = ge if any_ge is None else (any_ge | ge)
        zc = p_ref[0, :, :, 85 * a + 4]
        conf = jax.nn.sigmoid(zc)
        l1p = jnp.maximum(jnp.log1p(-conf), -100.0)
        lcn = lcn - jnp.sum(jnp.where(any_ge, 0.0, l1p))

    recs = []
    for (gx1, gy1, gx2, gy2, cx, cy, w, h, area, cf) in geo:
        rf = jnp.floor(cy)
        cc = jnp.floor(cx)
        r_i = rf.astype(jnp.int32)
        c_i = cc.astype(jnp.int32)
        best_a = jnp.int32(0)
        best_v = jnp.float32(-1.0)
        for a, (wa, ha) in enumerate(whs):
            acx = cc + 0.5
            acy = rf + 0.5
            ix = jnp.maximum(
                jnp.minimum(acx + wa * 0.5, gx2) - jnp.maximum(acx - wa * 0.5, gx1), 0.0)
            iy = jnp.maximum(
                jnp.minimum(acy + ha * 0.5, gy2) - jnp.maximum(acy - ha * 0.5, gy1), 0.0)
            inter = ix * iy
            union = jnp.maximum(wa * ha + area - inter, 1e-16)
            iou = inter / union
            take = iou > best_v
            best_a = jnp.where(take, jnp.int32(a), best_a)
            best_v = jnp.where(take, iou, best_v)
        hit = best_v >= _THR
        n_flat = (r_i * grid + c_i) * _A + best_a
        tx = cx - cc
        ty = cy - rf
        gscale = 2.0 - area / float(grid * grid)
        recs.append((r_i, c_i, best_a, n_flat, hit, tx, ty, w, h, gscale, cf))

    total = lcn
    for i, (r_i, c_i, a_i, n_i, hit_i, tx, ty, w, h, gscale, cf) in enumerate(recs):
        clobber = jnp.bool_(False)
        for j in range(i + 1, _NGT):
            clobber = clobber | (recs[j][4] & (recs[j][3] == n_i))
        live = hit_i & jnp.logical_not(clobber)
        v = p_ref[0, pl.ds(r_i, 1), pl.ds(c_i, 1), :]
        cls_i = (cf - 1.0).astype(jnp.int32)
        contrib = jnp.float32(0.0)
        for a, (wa, ha) in enumerate(whs):
            zx = v[0, 0, 85 * a + 0]
            zy = v[0, 0, 85 * a + 1]
            zw = v[0, 0, 85 * a + 2]
            zh = v[0, 0, 85 * a + 3]
            zc = v[0, 0, 85 * a + 4]
            logits = v[:, :, 85 * a + 5:85 * a + 85]
            tw = jnp.log(w / wa + 1e-16)
            th = jnp.log(h / ha + 1e-16)
            lxy = (jax.nn.sigmoid(zx) - tx) ** 2 + (jax.nn.sigmoid(zy) - ty) ** 2
            lwh = (jnp.tanh(zw) - tw) ** 2 + (jnp.tanh(zh) - th) ** 2
            m = jnp.max(logits)
            lse = m + jnp.log(jnp.sum(jnp.exp(logits - m)))
            sel = jnp.sum(jnp.where(
                jax.lax.broadcasted_iota(jnp.int32, (1, 1, _NC), 2) == cls_i, logits, 0.0))
            lcls = lse - sel
            conf = jax.nn.sigmoid(zc)
            lco = -jnp.maximum(jnp.log(conf), -100.0)
            term = gscale * (lxy + lwh) + lcls + lco
            contrib = jnp.where(a_i == a, term, contrib)
        total = total + jnp.where(live, contrib, 0.0)
    return total


def _yolo_kernel(p0_ref, p1_ref, p2_ref, gt_ref, out_ref):
    b = pl.program_id(0)
    gts = []
    for gi in range(_NGT):
        gts.append(tuple(gt_ref[0, gi, j] for j in range(5)))
    total = jnp.float32(0.0)
    for p_ref, grid, whs in ((p0_ref, 13, _WHS[0]),
                             (p1_ref, 26, _WHS[1]),
                             (p2_ref, 52, _WHS[2])):
        total = total + _scale_terms(p_ref, gts, grid, whs)

    @pl.when(b == 0)
    def _init():
        out_ref[0] = jnp.float32(0.0)

    out_ref[0] += total


@functools.partial(jax.jit, static_argnames=("interpret",))
def kernel(pred0, pred1, pred2, gt_bbox, interpret=False):
    B = pred0.shape[0]
    out = pl.pallas_call(
        _yolo_kernel,
        grid=(B,),
        in_specs=[
            pl.BlockSpec((1, 13, 13, 255), lambda b: (b, 0, 0, 0)),
            pl.BlockSpec((1, 26, 26, 255), lambda b: (b, 0, 0, 0)),
            pl.BlockSpec((1, 52, 52, 255), lambda b: (b, 0, 0, 0)),
            pl.BlockSpec((1, _NGT, 5), lambda b: (b, 0, 0),
                         memory_space=pltpu.SMEM),
        ],
        out_specs=pl.BlockSpec((1,), lambda b: (0,), memory_space=pltpu.SMEM),
        out_shape=jax.ShapeDtypeStruct((1,), jnp.float32),
        interpret=interpret,
    )(pred0, pred1, pred2, gt_bbox)
    return out


# speedup vs baseline: 15.3385x; 4.0006x over previous
import functools

import jax
import jax.numpy as jnp
from jax.experimental import pallas as pl
from jax.experimental.pallas import tpu as pltpu

_GRIDS = (13, 26, 52)
_A = 3
_NGT = 10
_NC = 80
_THR = 0.5
_WHS = (
    ((3.625, 2.8125), (4.875, 6.1875), (11.65625, 10.1875)),
    ((1.875, 3.8125), (3.875, 2.8125), (3.6875, 7.4375)),
    ((1.25, 1.625), (2.0, 3.75), (4.125, 2.875)),
)


def _scale_terms(p_ref, scratch_ref, gts, grid, whs):
    geo = []
    for (x1, y1, x2, y2, cf) in gts:
        gx1, gy1, gx2, gy2 = x1 * grid, y1 * grid, x2 * grid, y2 * grid
        cx = (gx1 + gx2) * 0.5
        cy = (gy1 + gy2) * 0.5
        w = gx2 - gx1
        h = gy2 - gy1
        area = w * h
        geo.append((gx1, gy1, gx2, gy2, cx, cy, w, h, area, cf))

    rows = jax.lax.broadcasted_iota(jnp.int32, (grid, grid), 0).astype(jnp.float32)
    cols = jax.lax.broadcasted_iota(jnp.int32, (grid, grid), 1).astype(jnp.float32)
    lcn = jnp.float32(0.0)
    for a, (wa, ha) in enumerate(whs):
        ax1 = cols + (0.5 - wa * 0.5)
        ax2 = cols + (0.5 + wa * 0.5)
        ay1 = rows + (0.5 - ha * 0.5)
        ay2 = rows + (0.5 + ha * 0.5)
        area_a = wa * ha
        ges = []
        for (gx1, gy1, gx2, gy2, cx, cy, w, h, area, cf) in geo:
            ix = jnp.maximum(jnp.minimum(ax2, gx2) - jnp.maximum(ax1, gx1), 0.0)
            iy = jnp.maximum(jnp.minimum(ay2, gy2) - jnp.maximum(ay1, gy1), 0.0)
            inter = ix * iy
            union = jnp.maximum(area_a + area - inter, 1e-16)
            ges.append((inter / union) >= _THR)
        while len(ges) > 1:
            ges = [a_ | b_ for a_, b_ in zip(ges[::2], ges[1::2])] + (
                [ges[-1]] if len(ges) % 2 else [])
        any_ge = ges[0]
        zc = p_ref[0, :, :, 85 * a + 4]
        conf = jax.nn.sigmoid(zc)
        l1p = jnp.maximum(jnp.log1p(-conf), -100.0)
        lcn = lcn - jnp.sum(jnp.where(any_ge, 0.0, l1p))

    recs = []
    for (gx1, gy1, gx2, gy2, cx, cy, w, h, area, cf) in geo:
        rf = jnp.floor(cy)
        cc = jnp.floor(cx)
        r_i = rf.astype(jnp.int32)
        c_i = cc.astype(jnp.int32)
        best_a = jnp.int32(0)
        best_v = jnp.float32(-1.0)
        for a, (wa, ha) in enumerate(whs):
            acx = cc + 0.5
            acy = rf + 0.5
            ix = jnp.maximum(
                jnp.minimum(acx + wa * 0.5, gx2) - jnp.maximum(acx - wa * 0.5, gx1), 0.0)
            iy = jnp.maximum(
                jnp.minimum(acy + ha * 0.5, gy2) - jnp.maximum(acy - ha * 0.5, gy1), 0.0)
            inter = ix * iy
            union = jnp.maximum(wa * ha + area - inter, 1e-16)
            iou = inter / union
            take = iou > best_v
            best_a = jnp.where(take, jnp.int32(a), best_a)
            best_v = jnp.where(take, iou, best_v)
        hit = best_v >= _THR
        n_flat = (r_i * grid + c_i) * _A + best_a
        tx = cx - cc
        ty = cy - rf
        gscale = 2.0 - area / float(grid * grid)
        recs.append((r_i, c_i, best_a, n_flat, hit, tx, ty, w, h, gscale, cf))

    lives = []
    for i, rec in enumerate(recs):
        clobber = jnp.bool_(False)
        for j in range(i + 1, _NGT):
            clobber = clobber | (recs[j][4] & (recs[j][3] == rec[3]))
        lives.append(rec[4] & jnp.logical_not(clobber))
    for i, rec in enumerate(recs):
        v = p_ref[0, pl.ds(rec[0], 1), pl.ds(rec[1], 1), :]
        scratch_ref[i:i + 1, :] = jnp.reshape(v, (1, 255))

    sub = jax.lax.broadcasted_iota(jnp.int32, (16, 1), 0)

    def chain(vals, dtype=jnp.float32):
        acc = jnp.zeros((16, 1), dtype)
        for i, s in enumerate(vals):
            acc = jnp.where(sub == i, s, acc)
        return acc

    tx16 = chain([r[5] for r in recs])
    ty16 = chain([r[6] for r in recs])
    w16 = chain([r[7] for r in recs])
    h16 = chain([r[8] for r in recs])
    gs16 = chain([r[9] for r in recs])
    cls16 = chain([(r[10] - 1.0).astype(jnp.int32) for r in recs], jnp.int32)

    rows = scratch_ref[:, :]
    lane80 = jax.lax.broadcasted_iota(jnp.int32, (16, _NC), 1)
    total = lcn
    for a, (wa, ha) in enumerate(whs):
        base = 85 * a
        mask16 = chain([(lives[i] & (recs[i][2] == a)).astype(jnp.int32)
                        for i in range(_NGT)], jnp.int32)
        zx = rows[:, base + 0:base + 1]
        zy = rows[:, base + 1:base + 2]
        zw = rows[:, base + 2:base + 3]
        zh = rows[:, base + 3:base + 4]
        zc = rows[:, base + 4:base + 5]
        logits = rows[:, base + 5:base + 85]
        m = jnp.max(logits, axis=1, keepdims=True)
        lse = m + jnp.log(jnp.sum(jnp.exp(logits - m), axis=1, keepdims=True))
        sel = jnp.sum(jnp.where(lane80 == cls16, logits, 0.0),
                      axis=1, keepdims=True)
        tw = jnp.log(w16 / wa + 1e-16)
        th = jnp.log(h16 / ha + 1e-16)
        lxy = (jax.nn.sigmoid(zx) - tx16) ** 2 + (jax.nn.sigmoid(zy) - ty16) ** 2
        lwh = (jnp.tanh(zw) - tw) ** 2 + (jnp.tanh(zh) - th) ** 2
        lco = -jnp.maximum(jnp.log(jax.nn.sigmoid(zc)), -100.0)
        term = gs16 * (lxy + lwh) + (lse - sel) + lco
        total = total + jnp.sum(jnp.where(mask16 != 0, term, 0.0))
    return total


def _yolo_kernel(p0_ref, p1_ref, p2_ref, gt_ref, out_ref, scratch_ref):
    b = pl.program_id(0)
    gts = []
    for gi in range(_NGT):
        gts.append(tuple(gt_ref[0, gi, j] for j in range(5)))
    total = jnp.float32(0.0)
    for p_ref, grid, whs in ((p0_ref, 13, _WHS[0]),
                             (p1_ref, 26, _WHS[1]),
                             (p2_ref, 52, _WHS[2])):
        total = total + _scale_terms(p_ref, scratch_ref, gts, grid, whs)

    @pl.when(b == 0)
    def _init():
        out_ref[0] = jnp.float32(0.0)

    out_ref[0] += total


@functools.partial(jax.jit, static_argnames=("interpret",))
def kernel(pred0, pred1, pred2, gt_bbox, interpret=False):
    B = pred0.shape[0]
    out = pl.pallas_call(
        _yolo_kernel,
        grid=(B,),
        in_specs=[
            pl.BlockSpec((1, 13, 13, 255), lambda b: (b, 0, 0, 0)),
            pl.BlockSpec((1, 26, 26, 255), lambda b: (b, 0, 0, 0)),
            pl.BlockSpec((1, 52, 52, 255), lambda b: (b, 0, 0, 0)),
            pl.BlockSpec((1, _NGT, 5), lambda b: (b, 0, 0),
                         memory_space=pltpu.SMEM),
        ],
        out_specs=pl.BlockSpec((1,), lambda b: (0,), memory_space=pltpu.SMEM),
        out_shape=jax.ShapeDtypeStruct((1,), jnp.float32),
        scratch_shapes=[pltpu.VMEM((16, 255), jnp.float32)],
        interpret=interpret,
    )(pred0, pred1, pred2, gt_bbox)
    return out
